# Initial kernel scaffold; baseline (speedup 1.0000x reference)
import jax
import jax.numpy as jnp
from jax.experimental import pallas as pl


def kernel(cent_obs, node_obs, adj, agent_id, W_embed, b_embed, Wg1, bg1, Wg2, bg2, gamma, beta, W1, b1, W2, b2, Wv, bv):
    raise NotImplementedError("write your pallas kernel here")


# speedup vs baseline: 2.5435x; 2.5435x over previous
import functools

import jax
import jax.numpy as jnp
from jax.experimental import pallas as pl
from jax.experimental.pallas import tpu as pltpu

B, N, DNODE, DCENT, H = 1024, 64, 128, 128, 256
MLP_IN = DCENT + H
BB = 16


def _body(node_ref, adj_ref, aid_ref, cent_ref,
          We_ref, be_ref, Wg1_ref, bg1_ref, Wg2_ref, bg2_ref,
          gam_ref, bet_ref, W1_ref, b1_ref, W2_ref, b2_ref, Wv_ref, bv_ref,
          out_ref):
    f32 = jnp.float32
    X = node_ref[...].reshape(BB * N, DNODE)
    h0 = jnp.maximum(
        jnp.dot(X, We_ref[...], preferred_element_type=f32) + be_ref[...], 0.0)
    adjb = adj_ref[...]
    deg = jnp.maximum(jnp.sum(adjb, axis=2, keepdims=True), 1e-6)
    A = adjb / deg
    g = jnp.dot(h0, Wg1_ref[...], preferred_element_type=f32)
    g3 = g.reshape(BB, N, H)
    m = jax.lax.dot_general(A, g3, (((2,), (1,)), ((0,), (0,))),
                            preferred_element_type=f32)
    h1 = jnp.maximum(m + bg1_ref[...], 0.0)
    aid2 = aid_ref[...]
    nidx = jax.lax.broadcasted_iota(jnp.int32, (BB, N), 1)
    onehotf = (nidx == aid2).astype(f32)
    arow = jax.lax.dot_general(onehotf, A, (((1,), (1,)), ((0,), (0,))),
                               preferred_element_type=f32)
    m2 = jax.lax.dot_general(arow, h1, (((1,), (1,)), ((0,), (0,))),
                             preferred_element_type=f32)
    feats = jnp.maximum(
        jnp.dot(m2, Wg2_ref[...], preferred_element_type=f32) + bg2_ref[...],
        0.0)
    inp = jnp.concatenate([cent_ref[...], feats], axis=1)
    mu = jnp.mean(inp, axis=1, keepdims=True)
    var = jnp.mean(inp * inp, axis=1, keepdims=True) - mu * mu
    x = (inp - mu) * jax.lax.rsqrt(var + 1e-5) * gam_ref[...] + bet_ref[...]
    x = jnp.maximum(jnp.dot(x, W1_ref[...], preferred_element_type=f32)
                    + b1_ref[...], 0.0)
    x = jnp.maximum(jnp.dot(x, W2_ref[...], preferred_element_type=f32)
                    + b2_ref[...], 0.0)
    out_ref[...] = jnp.dot(x, Wv_ref[...], preferred_element_type=f32) + bv_ref[...]


@functools.partial(jax.jit, static_argnames=())
def kernel(cent_obs, node_obs, adj, agent_id, W_embed, b_embed, Wg1, bg1,
           Wg2, bg2, gamma, beta, W1, b1, W2, b2, Wv, bv):
    nb = B // BB
    full = lambda shp: pl.BlockSpec(shp, lambda i: (0,) * len(shp))
    grid_spec = pl.GridSpec(
        grid=(nb,),
        in_specs=[
            pl.BlockSpec((BB, N, DNODE), lambda i: (i, 0, 0)),
            pl.BlockSpec((BB, N, N), lambda i: (i, 0, 0)),
            pl.BlockSpec((BB, 1), lambda i: (i, 0)),
            pl.BlockSpec((BB, DCENT), lambda i: (i, 0)),
            full((DNODE, H)), full((1, H)),
            full((H, H)), full((1, H)),
            full((H, H)), full((1, H)),
            full((1, MLP_IN)), full((1, MLP_IN)),
            full((MLP_IN, H)), full((1, H)),
            full((H, H)), full((1, H)),
            full((H, 1)), full((1, 1)),
        ],
        out_specs=pl.BlockSpec((BB, 1), lambda i: (i, 0)),
    )
    out = pl.pallas_call(
        _body,
        grid_spec=grid_spec,
        out_shape=jax.ShapeDtypeStruct((B, 1), jnp.float32),
        compiler_params=pltpu.CompilerParams(
            dimension_semantics=("parallel",)),
    )(node_obs, adj, agent_id.astype(jnp.int32), cent_obs,
      W_embed, b_embed.reshape(1, H),
      Wg1, bg1.reshape(1, H),
      Wg2, bg2.reshape(1, H),
      gamma.reshape(1, MLP_IN), beta.reshape(1, MLP_IN),
      W1, b1.reshape(1, H),
      W2, b2.reshape(1, H),
      Wv, bv.reshape(1, 1))
    return out
